# baseline (speedup 1.0000x reference)
import functools

import jax
import jax.numpy as jnp
from jax import lax
from jax.experimental import pallas as pl
from jax.experimental.pallas import tpu as pltpu
from jax.experimental.pallas import tpu_sc as plsc

N = 16384
W = 50
D = 32
V = 1000001
NB = 7813
VT = NB * 128
NC = 2
NS = 16
NW = NC * NS
UPT = N // NW
C = 8
CW = C * W
NBLK = UPT // 128
INV_W = 1.0 / W
MAGIC = 1311


def _lin_body(t4p_hbm, t4n_hbm, linp_hbm, linn_hbm,
              tc0, tc1, lb0, lb1, is0, is1, os0, os1):
    wid = lax.axis_index("s") * NC + lax.axis_index("c")
    iota16 = lax.iota(jnp.int32, 16)
    a_lo = lax.shift_right_logical(iota16, 3)
    a_hi = a_lo + 2
    r_v = iota16 & 7

    def transpose_col(tc, lb):
        for c in range(128):
            csp = iota16 * 0 + c
            lb[c, pl.ds(0, 16)] = plsc.load_gather(tc, [a_lo, r_v, csp])
            lb[c, pl.ds(16, 16)] = plsc.load_gather(tc, [a_hi, r_v, csp])

    def issue_in(src, b, tc, sem):
        def go():
            pltpu.async_copy(src.at[:, b], tc, sem)
        return go

    def wait_out(lb, dst, sem):
        def go():
            pltpu.make_async_copy(lb, dst.at[pl.ds(0, 128)], sem).wait()
        return go

    for src, dst in ((t4p_hbm, linp_hbm), (t4n_hbm, linn_hbm)):
        pltpu.async_copy(src.at[:, wid], tc0, is0)
        pltpu.async_copy(src.at[:, wid + 32], tc1, is1)

        def kbody(i, carry):
            b0 = i * 64 + wid
            b1 = b0 + 32
            pltpu.make_async_copy(src.at[:, 0], tc0, is0).wait()
            pl.when(i >= 1)(wait_out(lb0, dst, os0))
            transpose_col(tc0, lb0)
            pltpu.async_copy(lb0, dst.at[pl.ds(b0 * 128, 128)], os0)
            pl.when(b0 + 64 < NB)(issue_in(src, b0 + 64, tc0, is0))
            pltpu.make_async_copy(src.at[:, 0], tc1, is1).wait()
            pl.when(i >= 1)(wait_out(lb1, dst, os1))
            transpose_col(tc1, lb1)
            pltpu.async_copy(lb1, dst.at[pl.ds(b1 * 128, 128)], os1)
            pl.when(b1 + 64 < NB)(issue_in(src, b1 + 64, tc1, is1))
            return carry

        lax.fori_loop(0, 122, kbody, 0)

        def tail():
            pltpu.make_async_copy(src.at[:, 0], tc0, is0).wait()
            pltpu.make_async_copy(lb0, dst.at[pl.ds(0, 128)], os0).wait()
            transpose_col(tc0, lb0)
            pltpu.async_copy(lb0, dst.at[pl.ds((7808 + wid) * 128, 128)], os0)

        pl.when(wid < 5)(tail)
        wait_out(lb0, dst, os0)()
        wait_out(lb1, dst, os1)()


_sc_linearize = functools.partial(
    pl.kernel,
    out_type=(
        jax.ShapeDtypeStruct((VT, D), jnp.float32),
        jax.ShapeDtypeStruct((VT, D), jnp.float32),
    ),
    mesh=plsc.VectorSubcoreMesh(core_axis_name="c", subcore_axis_name="s"),
    scratch_types=[
        pltpu.VMEM((4, 8, 128), jnp.float32),
        pltpu.VMEM((4, 8, 128), jnp.float32),
        pltpu.VMEM((128, D), jnp.float32),
        pltpu.VMEM((128, D), jnp.float32),
        pltpu.SemaphoreType.DMA,
        pltpu.SemaphoreType.DMA,
        pltpu.SemaphoreType.DMA,
        pltpu.SemaphoreType.DMA,
    ],
    compiler_params=pltpu.CompilerParams(
        use_tc_tiling_on_sc=False, needs_layout_passes=False
    ),
)(_lin_body)


def _body(s4_hbm, pos_hbm, neg_hbm, out4_hbm,
          svb0, svb1, pc0, pc1, nc0, nc1, rows0, rows1, fm0, fm1,
          gsem0, gsem1, fsem0, fsem1, ssem):
    wid = lax.axis_index("s") * NC + lax.axis_index("c")
    ub0 = wid * NBLK

    iota16 = lax.iota(jnp.int32, 16)
    a_lo = lax.shift_right_logical(iota16, 3)
    r_v = iota16 & 7

    def load_state(b, svb):
        pltpu.async_copy(s4_hbm.at[:, ub0 + b], svb, ssem)

    def wait_state(svb):
        pltpu.make_async_copy(s4_hbm.at[:, 0], svb, ssem).wait()

    def build(svb, pc, nc, uc0):
        for i in range(CW // 16):
            j = iota16 + (i * 16)
            uu = lax.shift_right_logical(j * MAGIC, 16)
            w = j - uu * W
            t0 = uu * 0
            ucv = uu + uc0
            pc[pl.ds(i * 16, 16)] = plsc.load_gather(svb, [w, t0, ucv]) + 1
            nc[pl.ds(i * 16, 16)] = plsc.load_gather(svb, [w, t0 + 1, ucv]) + 1

    def fire(pc, nc, rows, sem):
        pltpu.async_copy(pos_hbm.at[pc], rows.at[pl.ds(0, CW)], sem)
        pltpu.async_copy(neg_hbm.at[nc], rows.at[pl.ds(CW, CW)], sem)

    def drain_gathers(rows, sem):
        pltpu.make_async_copy(pos_hbm.at[pl.ds(0, 2 * CW)], rows, sem).wait()

    def reduce_scatter(rows, fm, uc0):
        for uu in range(C):
            def rbody(r, accs):
                base = uu * W + r * 2
                a0, b0, a1, b1, a2, b2, a3, b3 = accs
                return (
                    a0 + rows[base, pl.ds(0, 16)],
                    b0 + rows[base, pl.ds(16, 16)],
                    a1 + rows[base + 1, pl.ds(0, 16)],
                    b1 + rows[base + 1, pl.ds(16, 16)],
                    a2 + rows[CW + base, pl.ds(0, 16)],
                    b2 + rows[CW + base, pl.ds(16, 16)],
                    a3 + rows[CW + base + 1, pl.ds(0, 16)],
                    b3 + rows[CW + base + 1, pl.ds(16, 16)],
                )

            z = jnp.zeros((16,), jnp.float32)
            a0, b0, a1, b1, a2, b2, a3, b3 = lax.fori_loop(
                0, W // 2, rbody, (z, z, z, z, z, z, z, z)
            )
            lo = ((a0 + a1) + (a2 + a3)) * INV_W
            hi = ((b0 + b1) + (b2 + b3)) * INV_W
            c_spl = iota16 * 0 + (uc0 + uu)
            plsc.store_scatter(fm, [a_lo, r_v, c_spl], lo)
            plsc.store_scatter(fm, [a_lo + 2, r_v, c_spl], hi)

    def flush(fm, b, fsem):
        pltpu.async_copy(fm, out4_hbm.at[:, ub0 + b], fsem)

    def wait_flush(fm, fsem):
        pltpu.make_async_copy(fm, out4_hbm.at[:, 0], fsem).wait()

    load_state(0, svb0)
    wait_state(svb0)
    build(svb0, pc0, nc0, 0)
    fire(pc0, nc0, rows0, gsem0)
    load_state(1, svb1)

    svb = (svb0, svb1)
    fm = (fm0, fm1)
    fsem = (fsem0, fsem1)

    for b in range(NBLK):
        p = b & 1
        svb_q = svb[b & 1]
        fm_p = fm[p]

        def ibody(ii, carry):
            uc0_0 = ii * 16
            uc0_1 = ii * 16 + 8
            build(svb_q, pc1, nc1, uc0_1)
            drain_gathers(rows0, gsem0)
            fire(pc1, nc1, rows1, gsem1)
            reduce_scatter(rows0, fm_p, uc0_0)
            pl.when(ii < 7)(lambda: build(svb_q, pc0, nc0, uc0_0 + 16))
            drain_gathers(rows1, gsem1)
            pl.when(ii < 7)(lambda: fire(pc0, nc0, rows0, gsem0))
            reduce_scatter(rows1, fm_p, uc0_1)
            return carry

        if b >= 2:
            wait_flush(fm_p, fsem[p])
        lax.fori_loop(0, 8, ibody, 0)
        flush(fm_p, b, fsem[p])
        if b < NBLK - 1:
            wait_state(svb[(b + 1) & 1])
            if b < NBLK - 2:
                load_state(b + 2, svb[b & 1])
            build(svb[(b + 1) & 1], pc0, nc0, 0)
            fire(pc0, nc0, rows0, gsem0)

    wait_flush(fm0, fsem0)
    wait_flush(fm1, fsem1)


_user_model_sc = functools.partial(
    pl.kernel,
    out_type=jax.ShapeDtypeStruct((4, 128, 8, 128), jnp.float32),
    mesh=plsc.VectorSubcoreMesh(core_axis_name="c", subcore_axis_name="s"),
    scratch_types=[
        pltpu.VMEM((W, 2, 128), jnp.int32),
        pltpu.VMEM((W, 2, 128), jnp.int32),
        pltpu.VMEM((CW,), jnp.int32),
        pltpu.VMEM((CW,), jnp.int32),
        pltpu.VMEM((CW,), jnp.int32),
        pltpu.VMEM((CW,), jnp.int32),
        pltpu.VMEM((2 * CW, D), jnp.float32),
        pltpu.VMEM((2 * CW, D), jnp.float32),
        pltpu.VMEM((4, 8, 128), jnp.float32),
        pltpu.VMEM((4, 8, 128), jnp.float32),
        pltpu.SemaphoreType.DMA,
        pltpu.SemaphoreType.DMA,
        pltpu.SemaphoreType.DMA,
        pltpu.SemaphoreType.DMA,
        pltpu.SemaphoreType.DMA,
    ],
    compiler_params=pltpu.CompilerParams(
        use_tc_tiling_on_sc=False, needs_layout_passes=False
    ),
)(_body)


def kernel(state, item_pos_emb, item_neg_emb):
    s4 = state.transpose(2, 1, 0).reshape(W, 2, 128, 128).transpose(0, 2, 1, 3)
    def t4(t):
        p = jnp.pad(t, ((0, VT - V), (0, 0)))
        return p.transpose(1, 0).reshape(4, 8, NB, 128).transpose(0, 2, 1, 3)

    linp, linn = _sc_linearize(t4(item_pos_emb), t4(item_neg_emb))
    out4 = _user_model_sc(s4, linp, linn)
    return out4.transpose(1, 3, 0, 2).reshape(N, D)


# speedup vs baseline: 1.0259x; 1.0259x over previous
import functools

import jax
import jax.numpy as jnp
from jax import lax
from jax.experimental import pallas as pl
from jax.experimental.pallas import tpu as pltpu
from jax.experimental.pallas import tpu_sc as plsc

N = 16384
W = 50
D = 32
V = 1000001
NB = 7936
VT = NB * 128
LBLK = NB // 4
NC = 2
NS = 16
NW = NC * NS
UPT = N // NW
C = 8
CW = C * W
NBLK = UPT // 128
INV_W = 1.0 / W
MAGIC = 1311


def _lin_body(t4p_hbm, t4n_hbm, linp_hbm, linn_hbm,
              tc0, tc1, lb0, lb1, is0, is1, os0, os1):
    wid = lax.axis_index("s") * NC + lax.axis_index("c")
    iota16 = lax.iota(jnp.int32, 16)
    a_lo = lax.shift_right_logical(iota16, 3)
    a_hi = a_lo + 2
    r_v = iota16 & 7

    def transpose_blk(tc, lb):
        for bb in range(4):
            bsp = iota16 * 0 + bb

            def cbody(ci, carry):
                for cu in range(4):
                    c = ci * 4 + cu
                    csp = iota16 * 0 + c
                    row = bb * 128 + c
                    lb[row, pl.ds(0, 16)] = plsc.load_gather(
                        tc, [a_lo, bsp, r_v, csp]
                    )
                    lb[row, pl.ds(16, 16)] = plsc.load_gather(
                        tc, [a_hi, bsp, r_v, csp]
                    )
                return carry

            lax.fori_loop(0, 32, cbody, 0)

    def issue_in(src, j, tc, sem):
        def go():
            pltpu.async_copy(src.at[:, pl.ds(j * 4, 4)], tc, sem)
        return go

    def wait_out(lb, dst, sem):
        def go():
            pltpu.make_async_copy(lb, dst.at[pl.ds(0, 512)], sem).wait()
        return go

    for src, dst in ((t4p_hbm, linp_hbm), (t4n_hbm, linn_hbm)):
        issue_in(src, wid, tc0, is0)()
        issue_in(src, wid + 32, tc1, is1)()

        def kbody(i, carry):
            j0 = i * 64 + wid
            j1 = j0 + 32
            pltpu.make_async_copy(src.at[:, pl.ds(0, 4)], tc0, is0).wait()
            pl.when(i >= 1)(wait_out(lb0, dst, os0))
            transpose_blk(tc0, lb0)
            pltpu.async_copy(lb0, dst.at[pl.ds(j0 * 512, 512)], os0)
            pl.when(i < 30)(issue_in(src, j0 + 64, tc0, is0))
            pltpu.make_async_copy(src.at[:, pl.ds(0, 4)], tc1, is1).wait()
            pl.when(i >= 1)(wait_out(lb1, dst, os1))
            transpose_blk(tc1, lb1)
            pltpu.async_copy(lb1, dst.at[pl.ds(j1 * 512, 512)], os1)
            pl.when(i < 30)(issue_in(src, j1 + 64, tc1, is1))
            return carry

        lax.fori_loop(0, 31, kbody, 0)
        wait_out(lb0, dst, os0)()
        wait_out(lb1, dst, os1)()


_sc_linearize = functools.partial(
    pl.kernel,
    out_type=(
        jax.ShapeDtypeStruct((VT, D), jnp.float32),
        jax.ShapeDtypeStruct((VT, D), jnp.float32),
    ),
    mesh=plsc.VectorSubcoreMesh(core_axis_name="c", subcore_axis_name="s"),
    scratch_types=[
        pltpu.VMEM((4, 4, 8, 128), jnp.float32),
        pltpu.VMEM((4, 4, 8, 128), jnp.float32),
        pltpu.VMEM((512, D), jnp.float32),
        pltpu.VMEM((512, D), jnp.float32),
        pltpu.SemaphoreType.DMA,
        pltpu.SemaphoreType.DMA,
        pltpu.SemaphoreType.DMA,
        pltpu.SemaphoreType.DMA,
    ],
    compiler_params=pltpu.CompilerParams(
        use_tc_tiling_on_sc=False, needs_layout_passes=False
    ),
)(_lin_body)


def _body(s4_hbm, pos_hbm, neg_hbm, out4_hbm,
          svb0, svb1, pc0, pc1, nc0, nc1, rows0, rows1, fm0, fm1,
          gsem0, gsem1, fsem0, fsem1, ssem):
    wid = lax.axis_index("s") * NC + lax.axis_index("c")
    ub0 = wid * NBLK

    iota16 = lax.iota(jnp.int32, 16)
    a_lo = lax.shift_right_logical(iota16, 3)
    r_v = iota16 & 7

    def load_state(b, svb):
        pltpu.async_copy(s4_hbm.at[:, ub0 + b], svb, ssem)

    def wait_state(svb):
        pltpu.make_async_copy(s4_hbm.at[:, 0], svb, ssem).wait()

    def build(svb, pc, nc, uc0):
        for i in range(CW // 16):
            j = iota16 + (i * 16)
            uu = lax.shift_right_logical(j * MAGIC, 16)
            w = j - uu * W
            t0 = uu * 0
            ucv = uu + uc0
            pc[pl.ds(i * 16, 16)] = plsc.load_gather(svb, [w, t0, ucv]) + 1
            nc[pl.ds(i * 16, 16)] = plsc.load_gather(svb, [w, t0 + 1, ucv]) + 1

    def fire(pc, nc, rows, sem):
        pltpu.async_copy(pos_hbm.at[pc], rows.at[pl.ds(0, CW)], sem)
        pltpu.async_copy(neg_hbm.at[nc], rows.at[pl.ds(CW, CW)], sem)

    def drain_gathers(rows, sem):
        pltpu.make_async_copy(pos_hbm.at[pl.ds(0, 2 * CW)], rows, sem).wait()

    def reduce_scatter(rows, fm, uc0):
        for uu in range(C):
            def rbody(r, accs):
                base = uu * W + r * 2
                a0, b0, a1, b1, a2, b2, a3, b3 = accs
                return (
                    a0 + rows[base, pl.ds(0, 16)],
                    b0 + rows[base, pl.ds(16, 16)],
                    a1 + rows[base + 1, pl.ds(0, 16)],
                    b1 + rows[base + 1, pl.ds(16, 16)],
                    a2 + rows[CW + base, pl.ds(0, 16)],
                    b2 + rows[CW + base, pl.ds(16, 16)],
                    a3 + rows[CW + base + 1, pl.ds(0, 16)],
                    b3 + rows[CW + base + 1, pl.ds(16, 16)],
                )

            z = jnp.zeros((16,), jnp.float32)
            a0, b0, a1, b1, a2, b2, a3, b3 = lax.fori_loop(
                0, W // 2, rbody, (z, z, z, z, z, z, z, z)
            )
            lo = ((a0 + a1) + (a2 + a3)) * INV_W
            hi = ((b0 + b1) + (b2 + b3)) * INV_W
            c_spl = iota16 * 0 + (uc0 + uu)
            plsc.store_scatter(fm, [a_lo, r_v, c_spl], lo)
            plsc.store_scatter(fm, [a_lo + 2, r_v, c_spl], hi)

    def flush(fm, b, fsem):
        pltpu.async_copy(fm, out4_hbm.at[:, ub0 + b], fsem)

    def wait_flush(fm, fsem):
        pltpu.make_async_copy(fm, out4_hbm.at[:, 0], fsem).wait()

    load_state(0, svb0)
    wait_state(svb0)
    build(svb0, pc0, nc0, 0)
    fire(pc0, nc0, rows0, gsem0)
    load_state(1, svb1)

    svb = (svb0, svb1)
    fm = (fm0, fm1)
    fsem = (fsem0, fsem1)

    for b in range(NBLK):
        p = b & 1
        svb_q = svb[b & 1]
        fm_p = fm[p]

        def ibody(ii, carry):
            uc0_0 = ii * 16
            uc0_1 = ii * 16 + 8
            build(svb_q, pc1, nc1, uc0_1)
            drain_gathers(rows0, gsem0)
            fire(pc1, nc1, rows1, gsem1)
            reduce_scatter(rows0, fm_p, uc0_0)
            pl.when(ii < 7)(lambda: build(svb_q, pc0, nc0, uc0_0 + 16))
            drain_gathers(rows1, gsem1)
            pl.when(ii < 7)(lambda: fire(pc0, nc0, rows0, gsem0))
            reduce_scatter(rows1, fm_p, uc0_1)
            return carry

        if b >= 2:
            wait_flush(fm_p, fsem[p])
        lax.fori_loop(0, 8, ibody, 0)
        flush(fm_p, b, fsem[p])
        if b < NBLK - 1:
            wait_state(svb[(b + 1) & 1])
            if b < NBLK - 2:
                load_state(b + 2, svb[b & 1])
            build(svb[(b + 1) & 1], pc0, nc0, 0)
            fire(pc0, nc0, rows0, gsem0)

    wait_flush(fm0, fsem0)
    wait_flush(fm1, fsem1)


_user_model_sc = functools.partial(
    pl.kernel,
    out_type=jax.ShapeDtypeStruct((4, 128, 8, 128), jnp.float32),
    mesh=plsc.VectorSubcoreMesh(core_axis_name="c", subcore_axis_name="s"),
    scratch_types=[
        pltpu.VMEM((W, 2, 128), jnp.int32),
        pltpu.VMEM((W, 2, 128), jnp.int32),
        pltpu.VMEM((CW,), jnp.int32),
        pltpu.VMEM((CW,), jnp.int32),
        pltpu.VMEM((CW,), jnp.int32),
        pltpu.VMEM((CW,), jnp.int32),
        pltpu.VMEM((2 * CW, D), jnp.float32),
        pltpu.VMEM((2 * CW, D), jnp.float32),
        pltpu.VMEM((4, 8, 128), jnp.float32),
        pltpu.VMEM((4, 8, 128), jnp.float32),
        pltpu.SemaphoreType.DMA,
        pltpu.SemaphoreType.DMA,
        pltpu.SemaphoreType.DMA,
        pltpu.SemaphoreType.DMA,
        pltpu.SemaphoreType.DMA,
    ],
    compiler_params=pltpu.CompilerParams(
        use_tc_tiling_on_sc=False, needs_layout_passes=False
    ),
)(_body)


def kernel(state, item_pos_emb, item_neg_emb):
    s4 = state.transpose(2, 1, 0).reshape(W, 2, 128, 128).transpose(0, 2, 1, 3)
    def t4(t):
        p = jnp.pad(t, ((0, VT - V), (0, 0)))
        return p.transpose(1, 0).reshape(4, 8, NB, 128).transpose(0, 2, 1, 3)


    linp, linn = _sc_linearize(t4(item_pos_emb), t4(item_neg_emb))
    out4 = _user_model_sc(s4, linp, linn)
    return out4.transpose(1, 3, 0, 2).reshape(N, D)
